# baseline (speedup 1.0000x reference)
import functools

import jax
import jax.numpy as jnp
from jax import lax
from jax.experimental import pallas as pl
from jax.experimental.pallas import tpu as pltpu
from jax.experimental.pallas import tpu_sc as plsc

N = 10000
E = 320000
D_IN = 128
D_MID = 256
D_OUT = 128

NP = 10240
NC = 2
NS = 16
CHUNK = 128

CA = -(-E // (NC * NS * CHUNK * 4)) * 4
EA = NC * NS * CA * CHUNK
CB = -(-E // (NS * CHUNK * 4)) * 4
EB = NS * CB * CHUNK

W = 128


def _sc_agg_body(x_hbm, src_hbm, dst_hbm, z_hbm, *rest, chunks, with_deg):
    if with_deg:
        out_hbm, deg_hbm = rest[:2]
        idx_s, idx_d, rows, hist, shared, sem = rest[2:]
    else:
        out_hbm = rest[0]
        idx_s, idx_d, rows, shared, sem = rest[1:]
        hist = None
    c = lax.axis_index("c")
    s = lax.axis_index("s")
    rps = NP // NS
    row0 = s * rps
    pltpu.sync_copy(z_hbm.at[pl.ds(row0, rps)], shared.at[pl.ds(row0, rps)])
    plsc.subcore_barrier()
    if with_deg:
        zeros16 = jnp.zeros((16,), jnp.float32)

        def zbody(i, carry):
            hist[pl.ds(i * 16, 16)] = zeros16
            return carry

        lax.fori_loop(0, NP // 16, zbody, 0)
    ones16 = jnp.ones((16,), jnp.float32)

    def chunk_body(i, carry):
        pltpu.sync_copy(src_hbm.at[c, s, i], idx_s)
        pltpu.sync_copy(dst_hbm.at[c, s, i], idx_d)
        pltpu.async_copy(x_hbm.at[idx_s], rows, sem).wait()
        pltpu.sync_copy(rows, shared.at[idx_d], add=True)
        if with_deg:
            for j in range(CHUNK // 16):
                iv = idx_d[pl.ds(j * 16, 16)]
                plsc.addupdate_scatter(hist, [iv], ones16)
        return carry

    lax.fori_loop(0, chunks, chunk_body, 0)
    plsc.subcore_barrier()
    pltpu.sync_copy(shared.at[pl.ds(row0, rps)], out_hbm.at[c, pl.ds(row0, rps)])
    if with_deg:
        pltpu.sync_copy(hist, deg_hbm.at[c * NS + s])


def _make_sc_agg(chunks, with_deg):
    out_type = [jax.ShapeDtypeStruct((NC, NP, W), jnp.float32)]
    scratch = [
        pltpu.VMEM((CHUNK,), jnp.int32),
        pltpu.VMEM((CHUNK,), jnp.int32),
        pltpu.VMEM((CHUNK, W), jnp.float32),
        pltpu.VMEM_SHARED((NP, W), jnp.float32),
        pltpu.SemaphoreType.DMA,
    ]
    if with_deg:
        out_type.append(jax.ShapeDtypeStruct((NC * NS, NP), jnp.float32))
        scratch.insert(3, pltpu.VMEM((NP,), jnp.float32))
    return pl.kernel(
        functools.partial(_sc_agg_body, chunks=chunks, with_deg=with_deg),
        out_type=tuple(out_type),
        mesh=plsc.VectorSubcoreMesh(core_axis_name="c", subcore_axis_name="s"),
        scratch_types=scratch,
        compiler_params=pltpu.CompilerParams(needs_layout_passes=False),
        name=f"sage_sc_agg_{chunks}_{int(with_deg)}",
    )


_sc_agg_a = _make_sc_agg(CA, True)
_sc_agg_b = _make_sc_agg(CB, False)
_sc_agg_c = _make_sc_agg(CA, False)


def _tc1_body(x_ref, p_ref, dg_ref, ws_ref, wn_ref, b_ref, o_ref):
    x = x_ref[...]
    agg = p_ref[0] + p_ref[1]
    deg = jnp.sum(dg_ref[...], axis=0)
    inv = 1.0 / jnp.maximum(deg, 1.0)
    h = (jnp.dot(x, ws_ref[...], preferred_element_type=jnp.float32)
         + jnp.dot(agg * inv[:, None], wn_ref[...], preferred_element_type=jnp.float32)
         + b_ref[...])
    o_ref[0] = jnp.maximum(h, 0.0)


def _tc2_body(x_ref, a_ref, dg_ref, ws1_ref, wn1_ref, b1_ref, ws2_ref, wn2_ref, b2_ref, s2_ref, p2_ref):
    xl, xh = x_ref[0], x_ref[1]
    al, ah = a_ref[0], a_ref[1]
    deg = jnp.sum(dg_ref[...], axis=0)
    inv = (1.0 / jnp.maximum(deg, 1.0))[:, None]
    dot = lambda a, b: jnp.dot(a, b, preferred_element_type=jnp.float32)
    h = (dot(xl, ws1_ref[:128]) + dot(xh, ws1_ref[128:])
         + dot(al * inv, wn1_ref[:128]) + dot(ah * inv, wn1_ref[128:])
         + b1_ref[...])
    h = jnp.maximum(h, 0.0)
    s2_ref[...] = dot(h, ws2_ref[...]) + b2_ref[...]
    p2_ref[...] = dot(h, wn2_ref[...])


def _tc3_body(s2_ref, p_ref, dg_ref, o_ref):
    deg = jnp.sum(dg_ref[...], axis=0)
    inv = (1.0 / jnp.maximum(deg, 1.0))[:, None]
    o_ref[...] = s2_ref[...] + (p_ref[0] + p_ref[1]) * inv


def kernel(features, edge_index, w_self0, w_neigh0, b0, w_self1, w_neigh1, b1, w_self2, w_neigh2, b2):
    f32 = jnp.float32
    src = edge_index[0].astype(jnp.int32)
    dst = edge_index[1].astype(jnp.int32)

    xpad = jnp.zeros((NP, W), f32)
    xpad = lax.dynamic_update_slice(xpad, features, (0, 0))

    pad_a = EA - E
    src_a = jnp.concatenate([src, jnp.full((pad_a,), N, jnp.int32)]).reshape(NC, NS, CA, CHUNK)
    dst_a = jnp.concatenate([dst, jnp.full((pad_a,), N, jnp.int32)]).reshape(NC, NS, CA, CHUNK)
    pad_b = EB - E
    src_b1 = jnp.concatenate([src, jnp.full((pad_b,), N, jnp.int32)]).reshape(NS, CB, CHUNK)
    src_b = jnp.stack([src_b1, src_b1 + NP])
    dst_b1 = jnp.concatenate([dst, jnp.full((pad_b,), N, jnp.int32)]).reshape(NS, CB, CHUNK)
    dst_b = jnp.stack([dst_b1, dst_b1])

    z_b = jnp.zeros((NP, W), f32)

    part0, degp = _sc_agg_a(xpad, src_a, dst_a, z_b)

    b0r = b0.reshape(1, D_MID)
    x1h = pl.pallas_call(
        _tc1_body,
        grid=(NP // 512, 2),
        in_specs=[
            pl.BlockSpec((512, 128), lambda g, h: (g, 0)),
            pl.BlockSpec((2, 512, 128), lambda g, h: (0, g, 0)),
            pl.BlockSpec((NC * NS, 512), lambda g, h: (0, g)),
            pl.BlockSpec((128, 128), lambda g, h: (0, h)),
            pl.BlockSpec((128, 128), lambda g, h: (0, h)),
            pl.BlockSpec((1, 128), lambda g, h: (0, h)),
        ],
        out_specs=pl.BlockSpec((1, 512, 128), lambda g, h: (h, g, 0)),
        out_shape=jax.ShapeDtypeStruct((2, NP, 128), f32),
    )(xpad, part0, degp, w_self0, w_neigh0, b0r)

    agg1, = _sc_agg_b(x1h.reshape(2 * NP, 128), src_b, dst_b, z_b)

    b1r = b1.reshape(1, D_MID)
    b2r = b2.reshape(1, D_OUT)
    s2, p2 = pl.pallas_call(
        _tc2_body,
        grid=(NP // 512,),
        in_specs=[
            pl.BlockSpec((2, 512, 128), lambda g: (0, g, 0)),
            pl.BlockSpec((2, 512, 128), lambda g: (0, g, 0)),
            pl.BlockSpec((NC * NS, 512), lambda g: (0, g)),
            pl.BlockSpec((D_MID, D_MID), lambda g: (0, 0)),
            pl.BlockSpec((D_MID, D_MID), lambda g: (0, 0)),
            pl.BlockSpec((1, D_MID), lambda g: (0, 0)),
            pl.BlockSpec((D_MID, D_OUT), lambda g: (0, 0)),
            pl.BlockSpec((D_MID, D_OUT), lambda g: (0, 0)),
            pl.BlockSpec((1, D_OUT), lambda g: (0, 0)),
        ],
        out_specs=[
            pl.BlockSpec((512, 128), lambda g: (g, 0)),
            pl.BlockSpec((512, 128), lambda g: (g, 0)),
        ],
        out_shape=[
            jax.ShapeDtypeStruct((NP, 128), f32),
            jax.ShapeDtypeStruct((NP, 128), f32),
        ],
    )(x1h, agg1, degp, w_self1, w_neigh1, b1r, w_self2, w_neigh2, b2r)

    part2, = _sc_agg_c(p2, src_a, dst_a, z_b)

    out = pl.pallas_call(
        _tc3_body,
        grid=(NP // 512,),
        in_specs=[
            pl.BlockSpec((512, 128), lambda g: (g, 0)),
            pl.BlockSpec((2, 512, 128), lambda g: (0, g, 0)),
            pl.BlockSpec((NC * NS, 512), lambda g: (0, g)),
        ],
        out_specs=pl.BlockSpec((512, 128), lambda g: (g, 0)),
        out_shape=jax.ShapeDtypeStruct((N, D_OUT), f32),
    )(s2, part2, degp)
    return out


# speedup vs baseline: 2.1284x; 2.1284x over previous
import functools

import jax
import jax.numpy as jnp
from jax import lax
from jax.experimental import pallas as pl
from jax.experimental.pallas import tpu as pltpu
from jax.experimental.pallas import tpu_sc as plsc

N = 10000
E = 320000
D_IN = 128
D_MID = 256
D_OUT = 128

NP = 10240
NC = 2
NS = 16
CHUNK = 128

CA = -(-E // (NC * NS * CHUNK))
EA = NC * NS * CA * CHUNK
CB = -(-E // (NS * CHUNK))
EB = NS * CB * CHUNK

W = 128


def _sc_agg_body(x_hbm, src_hbm, dst_hbm, z_hbm, *rest, chunks, with_deg):
    if with_deg:
        out_hbm, deg_hbm = rest[:2]
        idx_s, idx_d, rows, hist, shared, sem = rest[2:]
    else:
        out_hbm = rest[0]
        idx_s, idx_d, rows, shared, sem = rest[1:]
        hist = None
    c = lax.axis_index("c")
    s = lax.axis_index("s")
    rps = NP // NS
    row0 = s * rps
    pltpu.sync_copy(z_hbm.at[pl.ds(row0, rps)], shared.at[pl.ds(row0, rps)])
    plsc.subcore_barrier()
    if with_deg:
        zeros16 = jnp.zeros((16,), jnp.float32)

        def zbody(i, carry):
            hist[pl.ds(i * 16, 16)] = zeros16
            return carry

        lax.fori_loop(0, NP // 16, zbody, 0)
    ones16 = jnp.ones((16,), jnp.float32)

    def chunk_body(i, carry):
        pltpu.sync_copy(src_hbm.at[c, s, i], idx_s)
        pltpu.sync_copy(dst_hbm.at[c, s, i], idx_d)
        pltpu.async_copy(x_hbm.at[idx_s], rows, sem).wait()
        pltpu.sync_copy(rows, shared.at[idx_d], add=True)
        if with_deg:
            for j in range(CHUNK // 16):
                iv = idx_d[pl.ds(j * 16, 16)]
                plsc.addupdate_scatter(hist, [iv], ones16)
        return carry

    lax.fori_loop(0, chunks, chunk_body, 0)
    plsc.subcore_barrier()
    pltpu.sync_copy(shared.at[pl.ds(row0, rps)], out_hbm.at[c, pl.ds(row0, rps)])
    if with_deg:
        pltpu.sync_copy(hist, deg_hbm.at[c * NS + s])


def _make_sc_agg(chunks, with_deg):
    out_type = [jax.ShapeDtypeStruct((NC, NP, W), jnp.float32)]
    scratch = [
        pltpu.VMEM((CHUNK,), jnp.int32),
        pltpu.VMEM((CHUNK,), jnp.int32),
        pltpu.VMEM((CHUNK, W), jnp.float32),
        pltpu.VMEM_SHARED((NP, W), jnp.float32),
        pltpu.SemaphoreType.DMA,
    ]
    if with_deg:
        out_type.append(jax.ShapeDtypeStruct((NC * NS, NP), jnp.float32))
        scratch.insert(3, pltpu.VMEM((NP,), jnp.float32))
    return pl.kernel(
        functools.partial(_sc_agg_body, chunks=chunks, with_deg=with_deg),
        out_type=tuple(out_type),
        mesh=plsc.VectorSubcoreMesh(core_axis_name="c", subcore_axis_name="s"),
        scratch_types=scratch,
        compiler_params=pltpu.CompilerParams(needs_layout_passes=False),
        name=f"sage_sc_agg_{chunks}_{int(with_deg)}",
    )


_sc_agg_a = _make_sc_agg(CA, True)
_sc_agg_b = _make_sc_agg(CB, False)
_sc_agg_c = _make_sc_agg(CA, False)


def _tc1_body(x_ref, p_ref, dg_ref, ws_ref, wn_ref, b_ref, o_ref):
    x = x_ref[...]
    agg = p_ref[0] + p_ref[1]
    deg = jnp.sum(dg_ref[...], axis=0)
    inv = 1.0 / jnp.maximum(deg, 1.0)
    h = (jnp.dot(x, ws_ref[...], preferred_element_type=jnp.float32)
         + jnp.dot(agg * inv[:, None], wn_ref[...], preferred_element_type=jnp.float32)
         + b_ref[...])
    o_ref[0] = jnp.maximum(h, 0.0)


def _tc2_body(x_ref, a_ref, dg_ref, ws1_ref, wn1_ref, b1_ref, ws2_ref, wn2_ref, b2_ref, s2_ref, p2_ref):
    xl, xh = x_ref[0], x_ref[1]
    al, ah = a_ref[0], a_ref[1]
    deg = jnp.sum(dg_ref[...], axis=0)
    inv = (1.0 / jnp.maximum(deg, 1.0))[:, None]
    dot = lambda a, b: jnp.dot(a, b, preferred_element_type=jnp.float32)
    h = (dot(xl, ws1_ref[:128]) + dot(xh, ws1_ref[128:])
         + dot(al * inv, wn1_ref[:128]) + dot(ah * inv, wn1_ref[128:])
         + b1_ref[...])
    h = jnp.maximum(h, 0.0)
    s2_ref[...] = dot(h, ws2_ref[...]) + b2_ref[...]
    p2_ref[...] = dot(h, wn2_ref[...])


def _tc3_body(s2_ref, p_ref, dg_ref, o_ref):
    deg = jnp.sum(dg_ref[...], axis=0)
    inv = (1.0 / jnp.maximum(deg, 1.0))[:, None]
    o_ref[...] = s2_ref[...] + (p_ref[0] + p_ref[1]) * inv


def kernel(features, edge_index, w_self0, w_neigh0, b0, w_self1, w_neigh1, b1, w_self2, w_neigh2, b2):
    f32 = jnp.float32
    src = edge_index[0].astype(jnp.int32)
    dst = edge_index[1].astype(jnp.int32)

    xpad = jnp.zeros((NP, W), f32)
    xpad = lax.dynamic_update_slice(xpad, features, (0, 0))

    pad_a = EA - E
    fill_a = N + (jnp.arange(pad_a, dtype=jnp.int32) % (NP - N))
    src_a = jnp.concatenate([src, fill_a]).reshape(NC, NS, CA, CHUNK)
    dst_a = jnp.concatenate([dst, fill_a]).reshape(NC, NS, CA, CHUNK)
    pad_b = EB - E
    fill_b = N + (jnp.arange(pad_b, dtype=jnp.int32) % (NP - N))
    src_b1 = jnp.concatenate([src, fill_b]).reshape(NS, CB, CHUNK)
    src_b = jnp.stack([src_b1, src_b1 + NP])
    dst_b1 = jnp.concatenate([dst, fill_b]).reshape(NS, CB, CHUNK)
    dst_b = jnp.stack([dst_b1, dst_b1])

    z_b = jnp.zeros((NP, W), f32)

    part0, degp = _sc_agg_a(xpad, src_a, dst_a, z_b)

    b0r = b0.reshape(1, D_MID)
    x1h = pl.pallas_call(
        _tc1_body,
        grid=(NP // 512, 2),
        in_specs=[
            pl.BlockSpec((512, 128), lambda g, h: (g, 0)),
            pl.BlockSpec((2, 512, 128), lambda g, h: (0, g, 0)),
            pl.BlockSpec((NC * NS, 512), lambda g, h: (0, g)),
            pl.BlockSpec((128, 128), lambda g, h: (0, h)),
            pl.BlockSpec((128, 128), lambda g, h: (0, h)),
            pl.BlockSpec((1, 128), lambda g, h: (0, h)),
        ],
        out_specs=pl.BlockSpec((1, 512, 128), lambda g, h: (h, g, 0)),
        out_shape=jax.ShapeDtypeStruct((2, NP, 128), f32),
    )(xpad, part0, degp, w_self0, w_neigh0, b0r)

    agg1, = _sc_agg_b(x1h.reshape(2 * NP, 128), src_b, dst_b, z_b)

    b1r = b1.reshape(1, D_MID)
    b2r = b2.reshape(1, D_OUT)
    s2, p2 = pl.pallas_call(
        _tc2_body,
        grid=(NP // 512,),
        in_specs=[
            pl.BlockSpec((2, 512, 128), lambda g: (0, g, 0)),
            pl.BlockSpec((2, 512, 128), lambda g: (0, g, 0)),
            pl.BlockSpec((NC * NS, 512), lambda g: (0, g)),
            pl.BlockSpec((D_MID, D_MID), lambda g: (0, 0)),
            pl.BlockSpec((D_MID, D_MID), lambda g: (0, 0)),
            pl.BlockSpec((1, D_MID), lambda g: (0, 0)),
            pl.BlockSpec((D_MID, D_OUT), lambda g: (0, 0)),
            pl.BlockSpec((D_MID, D_OUT), lambda g: (0, 0)),
            pl.BlockSpec((1, D_OUT), lambda g: (0, 0)),
        ],
        out_specs=[
            pl.BlockSpec((512, 128), lambda g: (g, 0)),
            pl.BlockSpec((512, 128), lambda g: (g, 0)),
        ],
        out_shape=[
            jax.ShapeDtypeStruct((NP, 128), f32),
            jax.ShapeDtypeStruct((NP, 128), f32),
        ],
    )(x1h, agg1, degp, w_self1, w_neigh1, b1r, w_self2, w_neigh2, b2r)

    part2, = _sc_agg_c(p2, src_a, dst_a, z_b)

    out = pl.pallas_call(
        _tc3_body,
        grid=(NP // 512,),
        in_specs=[
            pl.BlockSpec((512, 128), lambda g: (g, 0)),
            pl.BlockSpec((2, 512, 128), lambda g: (0, g, 0)),
            pl.BlockSpec((NC * NS, 512), lambda g: (0, g)),
        ],
        out_specs=pl.BlockSpec((512, 128), lambda g: (g, 0)),
        out_shape=jax.ShapeDtypeStruct((N, D_OUT), f32),
    )(s2, part2, degp)
    return out


# speedup vs baseline: 3.2647x; 1.5339x over previous
import functools

import jax
import jax.numpy as jnp
from jax import lax
from jax.experimental import pallas as pl
from jax.experimental.pallas import tpu as pltpu
from jax.experimental.pallas import tpu_sc as plsc

N = 10000
E = 320000
D_IN = 128
D_MID = 256
D_OUT = 128

NP = 10240
NC = 2
NS = 16
CHUNK = 128

CA = -(-E // (NC * NS * CHUNK * 4)) * 4
EA = NC * NS * CA * CHUNK
CB = -(-E // (NS * CHUNK * 4)) * 4
EB = NS * CB * CHUNK

W = 128


def _sc_agg_body(x_hbm, src_hbm, dst_hbm, z_hbm, *rest, chunks, with_deg):
    if with_deg:
        out_hbm, deg_hbm = rest[:2]
        is0, id0, is1, id1, rows_a, rows_b, hist, shared, si0, si1, s0, s1 = rest[2:]
    else:
        out_hbm = rest[0]
        is0, id0, is1, id1, rows_a, rows_b, shared, si0, si1, s0, s1 = rest[1:]
        hist = None
    c = lax.axis_index("c")
    s = lax.axis_index("s")
    rps = NP // NS
    row0 = s * rps
    pltpu.sync_copy(z_hbm.at[pl.ds(row0, rps)], shared.at[pl.ds(row0, rps)])
    plsc.subcore_barrier()
    if with_deg:
        zeros16 = jnp.zeros((16,), jnp.float32)

        def zbody(i, carry):
            hist[pl.ds(i * 16, 16)] = zeros16
            return carry

        lax.fori_loop(0, NP // 16, zbody, 0)
    ones16 = jnp.ones((16,), jnp.float32)

    def hist_update(ib):
        if with_deg:
            for j in range(CHUNK // 16):
                iv = ib[pl.ds(j * 16, 16)]
                plsc.addupdate_scatter(hist, [iv], ones16)

    def gather(isrc, rows, sem):
        pltpu.async_copy(x_hbm.at[isrc], rows, sem)

    def gather_wait(isrc, rows, sem):
        pltpu.make_async_copy(x_hbm.at[isrc], rows, sem).wait()

    def scatter(idst, rows):
        pltpu.sync_copy(rows, shared.at[idst], add=True)

    def idx_load(p, ibs, ibd, sem):
        pltpu.async_copy(src_hbm.at[c, s, 2 * p], ibs.at[0], sem)
        pltpu.async_copy(src_hbm.at[c, s, 2 * p + 1], ibs.at[1], sem)
        pltpu.async_copy(dst_hbm.at[c, s, 2 * p], ibd.at[0], sem)
        pltpu.async_copy(dst_hbm.at[c, s, 2 * p + 1], ibd.at[1], sem)

    def idx_wait(p, ibs, ibd, sem):
        pltpu.make_async_copy(src_hbm.at[c, s, 2 * p], ibs.at[0], sem).wait()
        pltpu.make_async_copy(src_hbm.at[c, s, 2 * p + 1], ibs.at[1], sem).wait()
        pltpu.make_async_copy(dst_hbm.at[c, s, 2 * p], ibd.at[0], sem).wait()
        pltpu.make_async_copy(dst_hbm.at[c, s, 2 * p + 1], ibd.at[1], sem).wait()

    pairs = chunks // 2
    idx_load(0, is0, id0, si0)
    idx_load(1, is1, id1, si1)
    idx_wait(0, is0, id0, si0)
    gather(is0.at[0], rows_a, s0)

    def pair2_body(t, carry):
        p0 = 2 * t
        last = pairs // 2 - 1
        gather(is0.at[1], rows_b, s1)
        gather_wait(is0.at[0], rows_a, s0)
        scatter(id0.at[0], rows_a)
        hist_update(id0.at[0])
        gather_wait(is0.at[1], rows_b, s1)
        scatter(id0.at[1], rows_b)
        hist_update(id0.at[1])

        @pl.when(t < last)
        def _():
            idx_load(p0 + 2, is0, id0, si0)

        idx_wait(p0 + 1, is1, id1, si1)
        gather(is1.at[0], rows_a, s0)
        gather(is1.at[1], rows_b, s1)
        gather_wait(is1.at[0], rows_a, s0)
        scatter(id1.at[0], rows_a)
        hist_update(id1.at[0])
        gather_wait(is1.at[1], rows_b, s1)
        scatter(id1.at[1], rows_b)
        hist_update(id1.at[1])

        @pl.when(t < last)
        def _():
            idx_load(p0 + 3, is1, id1, si1)
            idx_wait(p0 + 2, is0, id0, si0)
            gather(is0.at[0], rows_a, s0)

        return carry

    lax.fori_loop(0, pairs // 2, pair2_body, 0)
    plsc.subcore_barrier()
    pltpu.sync_copy(shared.at[pl.ds(row0, rps)], out_hbm.at[c, pl.ds(row0, rps)])
    if with_deg:
        pltpu.sync_copy(hist, deg_hbm.at[c * NS + s])


def _make_sc_agg(chunks, with_deg):
    out_type = [jax.ShapeDtypeStruct((NC, NP, W), jnp.float32)]
    scratch = [
        pltpu.VMEM((2, CHUNK), jnp.int32),
        pltpu.VMEM((2, CHUNK), jnp.int32),
        pltpu.VMEM((2, CHUNK), jnp.int32),
        pltpu.VMEM((2, CHUNK), jnp.int32),
        pltpu.VMEM((CHUNK, W), jnp.float32),
        pltpu.VMEM((CHUNK, W), jnp.float32),
        pltpu.VMEM_SHARED((NP, W), jnp.float32),
        pltpu.SemaphoreType.DMA,
        pltpu.SemaphoreType.DMA,
        pltpu.SemaphoreType.DMA,
        pltpu.SemaphoreType.DMA,
    ]
    if with_deg:
        out_type.append(jax.ShapeDtypeStruct((NC * NS, NP), jnp.float32))
        scratch.insert(6, pltpu.VMEM((NP,), jnp.float32))
    return pl.kernel(
        functools.partial(_sc_agg_body, chunks=chunks, with_deg=with_deg),
        out_type=tuple(out_type),
        mesh=plsc.VectorSubcoreMesh(core_axis_name="c", subcore_axis_name="s"),
        scratch_types=scratch,
        compiler_params=pltpu.CompilerParams(needs_layout_passes=False),
        name=f"sage_sc_agg_{chunks}_{int(with_deg)}",
    )


_sc_agg_a = _make_sc_agg(CA, True)
_sc_agg_b = _make_sc_agg(CB, False)
_sc_agg_c = _make_sc_agg(CA, False)


def _tc1_body(x_ref, p_ref, dg_ref, ws_ref, wn_ref, b_ref, o_ref):
    x = x_ref[...]
    agg = p_ref[0] + p_ref[1]
    deg = jnp.sum(dg_ref[...], axis=0)
    inv = 1.0 / jnp.maximum(deg, 1.0)
    h = (jnp.dot(x, ws_ref[...], preferred_element_type=jnp.float32)
         + jnp.dot(agg * inv[:, None], wn_ref[...], preferred_element_type=jnp.float32)
         + b_ref[...])
    o_ref[0] = jnp.maximum(h, 0.0)


def _tc2_body(x_ref, a_ref, dg_ref, ws1_ref, wn1_ref, b1_ref, ws2_ref, wn2_ref, b2_ref, s2_ref, p2_ref):
    xl, xh = x_ref[0], x_ref[1]
    al, ah = a_ref[0], a_ref[1]
    deg = jnp.sum(dg_ref[...], axis=0)
    inv = (1.0 / jnp.maximum(deg, 1.0))[:, None]
    dot = lambda a, b: jnp.dot(a, b, preferred_element_type=jnp.float32)
    h = (dot(xl, ws1_ref[:128]) + dot(xh, ws1_ref[128:])
         + dot(al * inv, wn1_ref[:128]) + dot(ah * inv, wn1_ref[128:])
         + b1_ref[...])
    h = jnp.maximum(h, 0.0)
    s2_ref[...] = dot(h, ws2_ref[...]) + b2_ref[...]
    p2_ref[...] = dot(h, wn2_ref[...])


def _tc3_body(s2_ref, p_ref, dg_ref, o_ref):
    deg = jnp.sum(dg_ref[...], axis=0)
    inv = (1.0 / jnp.maximum(deg, 1.0))[:, None]
    o_ref[...] = s2_ref[...] + (p_ref[0] + p_ref[1]) * inv


def kernel(features, edge_index, w_self0, w_neigh0, b0, w_self1, w_neigh1, b1, w_self2, w_neigh2, b2):
    f32 = jnp.float32
    src = edge_index[0].astype(jnp.int32)
    dst = edge_index[1].astype(jnp.int32)

    xpad = jnp.zeros((NP, W), f32)
    xpad = lax.dynamic_update_slice(xpad, features, (0, 0))

    pad_a = EA - E
    fill_a = N + (jnp.arange(pad_a, dtype=jnp.int32) % (NP - N))
    src_a = jnp.concatenate([src, fill_a]).reshape(NC, NS, CA, CHUNK)
    dst_a = jnp.concatenate([dst, fill_a]).reshape(NC, NS, CA, CHUNK)
    pad_b = EB - E
    fill_b = N + (jnp.arange(pad_b, dtype=jnp.int32) % (NP - N))
    src_b1 = jnp.concatenate([src, fill_b]).reshape(NS, CB, CHUNK)
    src_b = jnp.stack([src_b1, src_b1 + NP])
    dst_b1 = jnp.concatenate([dst, fill_b]).reshape(NS, CB, CHUNK)
    dst_b = jnp.stack([dst_b1, dst_b1])

    z_b = jnp.zeros((NP, W), f32)

    part0, degp = _sc_agg_a(xpad, src_a, dst_a, z_b)

    b0r = b0.reshape(1, D_MID)
    x1h = pl.pallas_call(
        _tc1_body,
        grid=(NP // 512, 2),
        in_specs=[
            pl.BlockSpec((512, 128), lambda g, h: (g, 0)),
            pl.BlockSpec((2, 512, 128), lambda g, h: (0, g, 0)),
            pl.BlockSpec((NC * NS, 512), lambda g, h: (0, g)),
            pl.BlockSpec((128, 128), lambda g, h: (0, h)),
            pl.BlockSpec((128, 128), lambda g, h: (0, h)),
            pl.BlockSpec((1, 128), lambda g, h: (0, h)),
        ],
        out_specs=pl.BlockSpec((1, 512, 128), lambda g, h: (h, g, 0)),
        out_shape=jax.ShapeDtypeStruct((2, NP, 128), f32),
    )(xpad, part0, degp, w_self0, w_neigh0, b0r)

    agg1, = _sc_agg_b(x1h.reshape(2 * NP, 128), src_b, dst_b, z_b)

    b1r = b1.reshape(1, D_MID)
    b2r = b2.reshape(1, D_OUT)
    s2, p2 = pl.pallas_call(
        _tc2_body,
        grid=(NP // 512,),
        in_specs=[
            pl.BlockSpec((2, 512, 128), lambda g: (0, g, 0)),
            pl.BlockSpec((2, 512, 128), lambda g: (0, g, 0)),
            pl.BlockSpec((NC * NS, 512), lambda g: (0, g)),
            pl.BlockSpec((D_MID, D_MID), lambda g: (0, 0)),
            pl.BlockSpec((D_MID, D_MID), lambda g: (0, 0)),
            pl.BlockSpec((1, D_MID), lambda g: (0, 0)),
            pl.BlockSpec((D_MID, D_OUT), lambda g: (0, 0)),
            pl.BlockSpec((D_MID, D_OUT), lambda g: (0, 0)),
            pl.BlockSpec((1, D_OUT), lambda g: (0, 0)),
        ],
        out_specs=[
            pl.BlockSpec((512, 128), lambda g: (g, 0)),
            pl.BlockSpec((512, 128), lambda g: (g, 0)),
        ],
        out_shape=[
            jax.ShapeDtypeStruct((NP, 128), f32),
            jax.ShapeDtypeStruct((NP, 128), f32),
        ],
    )(x1h, agg1, degp, w_self1, w_neigh1, b1r, w_self2, w_neigh2, b2r)

    part2, = _sc_agg_c(p2, src_a, dst_a, z_b)

    out = pl.pallas_call(
        _tc3_body,
        grid=(NP // 512,),
        in_specs=[
            pl.BlockSpec((512, 128), lambda g: (g, 0)),
            pl.BlockSpec((2, 512, 128), lambda g: (0, g, 0)),
            pl.BlockSpec((NC * NS, 512), lambda g: (0, g)),
        ],
        out_specs=pl.BlockSpec((512, 128), lambda g: (g, 0)),
        out_shape=jax.ShapeDtypeStruct((N, D_OUT), f32),
    )(s2, part2, degp)
    return out


# speedup vs baseline: 3.8771x; 1.1876x over previous
import functools

import jax
import jax.numpy as jnp
from jax import lax
from jax.experimental import pallas as pl
from jax.experimental.pallas import tpu as pltpu
from jax.experimental.pallas import tpu_sc as plsc

N = 10000
E = 320000
D_IN = 128
D_MID = 256
D_OUT = 128

NP = 10240
NC = 2
NS = 16
CHUNK = 64

CA = -(-E // (NC * NS * CHUNK * 16)) * 16
EA = NC * NS * CA * CHUNK
CB = -(-E // (NS * CHUNK * 16)) * 16
EB = NS * CB * CHUNK

W = 128


def _sc_agg_body(x_hbm, src_hbm, dst_hbm, z_hbm, *rest, chunks, with_deg):
    if with_deg:
        out_hbm, deg_hbm = rest[:2]
        (ib0s, ib0d, ib1s, ib1d, r0, r1, r2, r3, hist, shared,
         si0, si1, g0, g1, g2, g3, sc0, sc1, sc2, sc3) = rest[2:]
    else:
        out_hbm = rest[0]
        (ib0s, ib0d, ib1s, ib1d, r0, r1, r2, r3, shared,
         si0, si1, g0, g1, g2, g3, sc0, sc1, sc2, sc3) = rest[1:]
        hist = None
    rows = [r0, r1, r2, r3]
    gsem = [g0, g1, g2, g3]
    ssem = [sc0, sc1, sc2, sc3]
    c = lax.axis_index("c")
    s = lax.axis_index("s")
    rps = NP // NS
    row0 = s * rps
    nbody = chunks // 16
    pltpu.sync_copy(z_hbm.at[pl.ds(row0, rps)], shared.at[pl.ds(row0, rps)])
    plsc.subcore_barrier()
    if with_deg:
        zeros16 = jnp.zeros((16,), jnp.float32)

        def zbody(i, carry):
            hist[pl.ds(i * 16, 16)] = zeros16
            return carry

        lax.fori_loop(0, NP // 16, zbody, 0)
    ones16 = jnp.ones((16,), jnp.float32)

    def hist_update(bd, e):
        if with_deg:
            for j in range(CHUNK // 16):
                iv = bd[e, pl.ds(j * 16, 16)]
                plsc.addupdate_scatter(hist, [iv], ones16)

    def gather_issue(bs, e, k):
        pltpu.async_copy(x_hbm.at[bs.at[e]], rows[k], gsem[k])

    def gather_wait(bs, e, k):
        pltpu.make_async_copy(x_hbm.at[bs.at[e]], rows[k], gsem[k]).wait()

    def scat_issue(bd, e, k):
        pltpu.async_copy(rows[k], shared.at[bd.at[e]], ssem[k], add=True)

    def scat_wait(bd, e, k):
        pltpu.make_async_copy(rows[k], shared.at[bd.at[e]], ssem[k]).wait()

    def idx_load(blk, bs, bd, sem):
        pltpu.async_copy(src_hbm.at[c, s, blk], bs, sem)
        pltpu.async_copy(dst_hbm.at[c, s, blk], bd, sem)

    def idx_wait(blk, bs, bd, sem):
        pltpu.make_async_copy(src_hbm.at[c, s, blk], bs, sem).wait()
        pltpu.make_async_copy(dst_hbm.at[c, s, blk], bd, sem).wait()

    def entry(q):
        if q < 8:
            return ib0s, ib0d, q
        if q < 16:
            return ib1s, ib1d, q - 8
        return ib0s, ib0d, q - 16

    idx_load(0, ib0s, ib0d, si0)
    idx_wait(0, ib0s, ib0d, si0)
    gather_issue(ib0s, 0, 0)
    gather_issue(ib0s, 1, 1)

    def block_body(b, carry):
        for q in range(16):
            bs, bd, e = entry(q)
            k = q % 4
            gather_wait(bs, e, k)
            if q >= 2:
                _, pd, pe = entry(q - 2)
                scat_wait(pd, pe, (q - 2) % 4)
            else:
                @pl.when(b > 0)
                def _(pe=6 + q, pk=(q + 2) % 4):
                    scat_wait(ib1d, pe, pk)
            scat_issue(bd, e, k)
            hist_update(bd, e)
            if q == 1:
                idx_load(2 * b + 1, ib1s, ib1d, si1)
            if q == 9:
                @pl.when(b < nbody - 1)
                def _():
                    idx_load(2 * b + 2, ib0s, ib0d, si0)
            qq = q + 2
            gs, gd2, ge = entry(qq)
            if qq < 16:
                if q == 6:
                    idx_wait(2 * b + 1, ib1s, ib1d, si1)
                gather_issue(gs, ge, qq % 4)
            elif qq == 16:
                @pl.when(b < nbody - 1)
                def _():
                    idx_wait(2 * b + 2, ib0s, ib0d, si0)
                    gather_issue(ib0s, 0, 0)
            else:
                @pl.when(b < nbody - 1)
                def _():
                    gather_issue(ib0s, 1, 1)
        return carry

    lax.fori_loop(0, nbody, block_body, 0)
    scat_wait(ib1d, 6, 2)
    scat_wait(ib1d, 7, 3)
    plsc.subcore_barrier()
    pltpu.sync_copy(shared.at[pl.ds(row0, rps)], out_hbm.at[c, pl.ds(row0, rps)])
    if with_deg:
        pltpu.sync_copy(hist, deg_hbm.at[c * NS + s])


def _make_sc_agg(chunks, with_deg):
    out_type = [jax.ShapeDtypeStruct((NC, NP, W), jnp.float32)]
    scratch = [
        pltpu.VMEM((8, CHUNK), jnp.int32),
        pltpu.VMEM((8, CHUNK), jnp.int32),
        pltpu.VMEM((8, CHUNK), jnp.int32),
        pltpu.VMEM((8, CHUNK), jnp.int32),
        pltpu.VMEM((CHUNK, W), jnp.float32),
        pltpu.VMEM((CHUNK, W), jnp.float32),
        pltpu.VMEM((CHUNK, W), jnp.float32),
        pltpu.VMEM((CHUNK, W), jnp.float32),
        pltpu.VMEM_SHARED((NP, W), jnp.float32),
        pltpu.SemaphoreType.DMA,
        pltpu.SemaphoreType.DMA,
        pltpu.SemaphoreType.DMA,
        pltpu.SemaphoreType.DMA,
        pltpu.SemaphoreType.DMA,
        pltpu.SemaphoreType.DMA,
        pltpu.SemaphoreType.DMA,
        pltpu.SemaphoreType.DMA,
        pltpu.SemaphoreType.DMA,
        pltpu.SemaphoreType.DMA,
    ]
    if with_deg:
        out_type.append(jax.ShapeDtypeStruct((NC * NS, NP), jnp.float32))
        scratch.insert(8, pltpu.VMEM((NP,), jnp.float32))
    return pl.kernel(
        functools.partial(_sc_agg_body, chunks=chunks, with_deg=with_deg),
        out_type=tuple(out_type),
        mesh=plsc.VectorSubcoreMesh(core_axis_name="c", subcore_axis_name="s"),
        scratch_types=scratch,
        compiler_params=pltpu.CompilerParams(needs_layout_passes=False),
        name=f"sage_sc_agg_{chunks}_{int(with_deg)}",
    )


_sc_agg_a = _make_sc_agg(CA, True)
_sc_agg_b = _make_sc_agg(CB, False)
_sc_agg_c = _make_sc_agg(CA, False)


def _tc1_body(x_ref, p_ref, dg_ref, ws_ref, wn_ref, b_ref, o_ref):
    x = x_ref[...]
    agg = p_ref[0] + p_ref[1]
    deg = jnp.sum(dg_ref[...], axis=0)
    inv = 1.0 / jnp.maximum(deg, 1.0)
    h = (jnp.dot(x, ws_ref[...], preferred_element_type=jnp.float32)
         + jnp.dot(agg * inv[:, None], wn_ref[...], preferred_element_type=jnp.float32)
         + b_ref[...])
    o_ref[0] = jnp.maximum(h, 0.0)


def _tc2_body(x_ref, a_ref, dg_ref, ws1_ref, wn1_ref, b1_ref, ws2_ref, wn2_ref, b2_ref, s2_ref, p2_ref):
    xl, xh = x_ref[0], x_ref[1]
    al, ah = a_ref[0], a_ref[1]
    deg = jnp.sum(dg_ref[...], axis=0)
    inv = (1.0 / jnp.maximum(deg, 1.0))[:, None]
    dot = lambda a, b: jnp.dot(a, b, preferred_element_type=jnp.float32)
    h = (dot(xl, ws1_ref[:128]) + dot(xh, ws1_ref[128:])
         + dot(al * inv, wn1_ref[:128]) + dot(ah * inv, wn1_ref[128:])
         + b1_ref[...])
    h = jnp.maximum(h, 0.0)
    s2_ref[...] = dot(h, ws2_ref[...]) + b2_ref[...]
    p2_ref[...] = dot(h, wn2_ref[...])


def _tc3_body(s2_ref, p_ref, dg_ref, o_ref):
    deg = jnp.sum(dg_ref[...], axis=0)
    inv = (1.0 / jnp.maximum(deg, 1.0))[:, None]
    o_ref[...] = s2_ref[...] + (p_ref[0] + p_ref[1]) * inv


def kernel(features, edge_index, w_self0, w_neigh0, b0, w_self1, w_neigh1, b1, w_self2, w_neigh2, b2):
    f32 = jnp.float32
    src = edge_index[0].astype(jnp.int32)
    dst = edge_index[1].astype(jnp.int32)

    xpad = jnp.zeros((NP, W), f32)
    xpad = lax.dynamic_update_slice(xpad, features, (0, 0))

    pad_a = EA - E
    fill_a = N + (jnp.arange(pad_a, dtype=jnp.int32) % (NP - N))
    src_a = jnp.concatenate([src, fill_a]).reshape(NC, NS, CA // 8, 8, CHUNK)
    dst_a = jnp.concatenate([dst, fill_a]).reshape(NC, NS, CA // 8, 8, CHUNK)
    pad_b = EB - E
    fill_b = N + (jnp.arange(pad_b, dtype=jnp.int32) % (NP - N))
    src_b1 = jnp.concatenate([src, fill_b]).reshape(NS, CB // 8, 8, CHUNK)
    src_b = jnp.stack([src_b1, src_b1 + NP])
    dst_b1 = jnp.concatenate([dst, fill_b]).reshape(NS, CB // 8, 8, CHUNK)
    dst_b = jnp.stack([dst_b1, dst_b1])

    z_b = jnp.zeros((NP, W), f32)

    part0, degp = _sc_agg_a(xpad, src_a, dst_a, z_b)

    b0r = b0.reshape(1, D_MID)
    x1h = pl.pallas_call(
        _tc1_body,
        grid=(NP // 512, 2),
        in_specs=[
            pl.BlockSpec((512, 128), lambda g, h: (g, 0)),
            pl.BlockSpec((2, 512, 128), lambda g, h: (0, g, 0)),
            pl.BlockSpec((NC * NS, 512), lambda g, h: (0, g)),
            pl.BlockSpec((128, 128), lambda g, h: (0, h)),
            pl.BlockSpec((128, 128), lambda g, h: (0, h)),
            pl.BlockSpec((1, 128), lambda g, h: (0, h)),
        ],
        out_specs=pl.BlockSpec((1, 512, 128), lambda g, h: (h, g, 0)),
        out_shape=jax.ShapeDtypeStruct((2, NP, 128), f32),
    )(xpad, part0, degp, w_self0, w_neigh0, b0r)

    agg1, = _sc_agg_b(x1h.reshape(2 * NP, 128), src_b, dst_b, z_b)

    b1r = b1.reshape(1, D_MID)
    b2r = b2.reshape(1, D_OUT)
    s2, p2 = pl.pallas_call(
        _tc2_body,
        grid=(NP // 512,),
        in_specs=[
            pl.BlockSpec((2, 512, 128), lambda g: (0, g, 0)),
            pl.BlockSpec((2, 512, 128), lambda g: (0, g, 0)),
            pl.BlockSpec((NC * NS, 512), lambda g: (0, g)),
            pl.BlockSpec((D_MID, D_MID), lambda g: (0, 0)),
            pl.BlockSpec((D_MID, D_MID), lambda g: (0, 0)),
            pl.BlockSpec((1, D_MID), lambda g: (0, 0)),
            pl.BlockSpec((D_MID, D_OUT), lambda g: (0, 0)),
            pl.BlockSpec((D_MID, D_OUT), lambda g: (0, 0)),
            pl.BlockSpec((1, D_OUT), lambda g: (0, 0)),
        ],
        out_specs=[
            pl.BlockSpec((512, 128), lambda g: (g, 0)),
            pl.BlockSpec((512, 128), lambda g: (g, 0)),
        ],
        out_shape=[
            jax.ShapeDtypeStruct((NP, 128), f32),
            jax.ShapeDtypeStruct((NP, 128), f32),
        ],
    )(x1h, agg1, degp, w_self1, w_neigh1, b1r, w_self2, w_neigh2, b2r)

    part2, = _sc_agg_c(p2, src_a, dst_a, z_b)

    out = pl.pallas_call(
        _tc3_body,
        grid=(NP // 512,),
        in_specs=[
            pl.BlockSpec((512, 128), lambda g: (g, 0)),
            pl.BlockSpec((2, 512, 128), lambda g: (0, g, 0)),
            pl.BlockSpec((NC * NS, 512), lambda g: (0, g)),
        ],
        out_specs=pl.BlockSpec((512, 128), lambda g: (g, 0)),
        out_shape=jax.ShapeDtypeStruct((N, D_OUT), f32),
    )(s2, part2, degp)
    return out
